# baseline (speedup 1.0000x reference)
import functools

import jax
import jax.numpy as jnp
from jax import lax
from jax.experimental import pallas as pl
from jax.experimental.pallas import tpu as pltpu
from jax.experimental.pallas import tpu_sc as plsc

_NUM_EMB = 1000000
_D = 32
_NW = 32
_L = 16


@functools.lru_cache(maxsize=None)
def _build(T, J):
    B = T * J
    TPW = T // _NW
    BPW = B // _NW
    NCH = J
    CC = TPW // 128
    NR = _D // 8
    assert TPW % 128 == 0 and NCH % 2 == 0 and NCH >= 6

    mesh = plsc.VectorSubcoreMesh(core_axis_name="c", subcore_axis_name="s")

    @functools.partial(
        pl.kernel,
        out_type=jax.ShapeDtypeStruct((J, NR, T // 128, 8, 128), jnp.float32),
        mesh=mesh,
        scratch_types=[
            pltpu.VMEM((BPW,), jnp.int32),
            *[pltpu.VMEM((TPW, _D), jnp.float32) for _ in range(2)],
            *[pltpu.VMEM((NR, CC, 8, 128), jnp.float32) for _ in range(2)],
            *[pltpu.SemaphoreType.DMA for _ in range(4)],
        ],
        compiler_params=pltpu.CompilerParams(
            use_tc_tiling_on_sc=False, needs_layout_passes=False
        ),
    )
    def k(x_hbm, tab_hbm, out_hbm, idx_v, *rest):
        bufs = rest[0:2]
        zbufs = rest[2:4]
        gsems = rest[4:6]
        wsems = rest[6:8]

        wid = lax.axis_index("s") * 2 + lax.axis_index("c")
        base = wid * BPW
        pltpu.sync_copy(x_hbm.at[pl.ds(base, BPW)], idx_v)

        lane = lax.iota(jnp.int32, 16)

        def clamp_chunk(c):
            def body(i, carry):
                sl = pl.ds(c * TPW + i * _L, _L)
                v = idx_v[sl]
                idx_v[sl] = jnp.minimum(jnp.maximum(v, 0), _NUM_EMB - 1)
                return carry

            lax.fori_loop(0, TPW // _L, body, 0)

        def fire_gather(c, b):
            clamp_chunk(c)
            return pltpu.async_copy(
                tab_hbm.at[idx_v.at[pl.ds(c * TPW, TPW)]], bufs[b], gsems[b]
            )

        def wait_gather(b):
            pltpu.make_async_copy(
                tab_hbm.at[idx_v.at[pl.ds(0, TPW)]], bufs[b], gsems[b]
            ).wait()

        def fire_writes(c, zb):
            return [
                pltpu.async_copy(
                    zbufs[zb].at[r],
                    out_hbm.at[c, r, pl.ds(wid * CC, CC)],
                    wsems[zb],
                )
                for r in range(NR)
            ]

        def wait_writes(zb):
            for r in range(NR):
                pltpu.make_async_copy(
                    zbufs[zb].at[r], out_hbm.at[0, r, pl.ds(0, CC)], wsems[zb]
                ).wait()

        def transpose_chunk(b, zb):
            rows = bufs[b]
            z = zbufs[zb]

            def body(i, carry):
                cc = i // 8
                s = i % 8
                for r in range(NR):
                    idx_d = jnp.full((16,), 8 * r + s, jnp.int32)
                    for m in range(8):
                        idx_t = cc * 128 + 16 * m + lane
                        z[r, cc, s, pl.ds(16 * m, 16)] = plsc.load_gather(
                            rows, [idx_t, idx_d]
                        )
                return carry

            lax.fori_loop(0, CC * 8, body, 0)

        fire_gather(0, 0)
        fire_gather(1, 1)
        for c in (0, 1):
            wait_gather(c)
            transpose_chunk(c, c)
            fire_gather(c + 2, c)
            fire_writes(c, c)

        def round_body(p, carry):
            for b in (0, 1):
                c = 2 * p + b
                wait_gather(b)
                wait_writes(b)
                transpose_chunk(b, b)

                @pl.when(c + 2 < NCH)
                def _():
                    fire_gather(c + 2, b)

                fire_writes(c, b)
            return carry

        lax.fori_loop(1, NCH // 2, round_body, 0)

        for zb in (0, 1):
            wait_writes(zb)

    return k


def kernel(x, embedding_table):
    T, J = x.shape
    xp = jnp.transpose(
        jnp.reshape(x.astype(jnp.int32), (_NW, T // _NW, J)), (0, 2, 1)
    ).reshape(-1)
    rr = lax.optimization_barrier(
        jnp.reshape(embedding_table, (_NUM_EMB * _D // 128, 128))
    )
    rt = jnp.reshape(rr, (_NUM_EMB, _D))
    z = _build(T, J)(xp, rt)
    return jnp.transpose(z, (2, 4, 0, 1, 3)).reshape(T, J, _D)


# speedup vs baseline: 1.0637x; 1.0637x over previous
import functools

import jax
import jax.numpy as jnp
from jax import lax
from jax.experimental import pallas as pl
from jax.experimental.pallas import tpu as pltpu
from jax.experimental.pallas import tpu_sc as plsc

_NUM_EMB = 1000000
_D = 32
_NW = 32
_L = 16
_NBUF = 6
_DEPTH = 4


@functools.lru_cache(maxsize=None)
def _build_permute(T, J):
    TPW = T // _NW
    BPW = TPW * J
    mesh = plsc.VectorSubcoreMesh(core_axis_name="c", subcore_axis_name="s")

    @functools.partial(
        pl.kernel,
        out_type=jax.ShapeDtypeStruct((T * J,), jnp.int32),
        mesh=mesh,
        scratch_types=[
            pltpu.VMEM((J, TPW), jnp.int32),
            pltpu.VMEM((BPW,), jnp.int32),
        ],
        compiler_params=pltpu.CompilerParams(
            use_tc_tiling_on_sc=True, needs_layout_passes=False
        ),
    )
    def k0(xt_hbm, xp_hbm, slab, flat):
        wid = lax.axis_index("s") * 2 + lax.axis_index("c")
        pltpu.sync_copy(xt_hbm.at[:, pl.ds(wid * TPW, TPW)], slab)

        def body(i, carry):
            j = i // (TPW // _L)
            g = i % (TPW // _L)
            v = slab[j, pl.ds(g * _L, _L)]
            v = jnp.minimum(jnp.maximum(v, 0), _NUM_EMB - 1)
            flat[pl.ds(j * TPW + g * _L, _L)] = v
            return carry

        lax.fori_loop(0, J * (TPW // _L), body, 0)
        pltpu.sync_copy(flat, xp_hbm.at[pl.ds(wid * BPW, BPW)])

    return k0


@functools.lru_cache(maxsize=None)
def _build_gather(T, J):
    B = T * J
    TPW = T // _NW
    BPW = B // _NW
    NCH = J
    assert TPW % _L == 0 and TPW % 8 == 0 and NCH > _NBUF

    mesh = plsc.VectorSubcoreMesh(core_axis_name="c", subcore_axis_name="s")

    @functools.partial(
        pl.kernel,
        out_type=jax.ShapeDtypeStruct((T, J, _D), jnp.float32),
        mesh=mesh,
        scratch_types=[
            pltpu.VMEM((BPW,), jnp.int32),
            *[pltpu.VMEM((TPW, _D), jnp.float32) for _ in range(_NBUF)],
            *[pltpu.SemaphoreType.DMA for _ in range(2 * _NBUF)],
        ],
        compiler_params=pltpu.CompilerParams(
            use_tc_tiling_on_sc=False, needs_layout_passes=False
        ),
    )
    def k1(x_hbm, tab_hbm, out_hbm, idx_v, *rest):
        bufs = rest[:_NBUF]
        gsems = rest[_NBUF:2 * _NBUF]
        wsems = rest[2 * _NBUF:]

        wid = lax.axis_index("s") * 2 + lax.axis_index("c")
        base = wid * BPW
        t_base = wid * TPW
        pltpu.sync_copy(x_hbm.at[pl.ds(base, BPW)], idx_v)

        gd = [None] * _NBUF
        wd = [None] * _NBUF

        def fire_gather(c):
            b = c % _NBUF
            gd[b] = pltpu.async_copy(
                tab_hbm.at[idx_v.at[pl.ds(c * TPW, TPW)]], bufs[b], gsems[b]
            )

        for j in range(_DEPTH):
            fire_gather(j)
        for c in range(NCH):
            b = c % _NBUF
            if c + _DEPTH < NCH:
                pb = (c + _DEPTH) % _NBUF
                if c + _DEPTH - _NBUF >= 0:
                    wd[pb].wait()
                fire_gather(c + _DEPTH)
            gd[b].wait()
            wd[b] = pltpu.async_copy(
                bufs[b], out_hbm.at[pl.ds(t_base, TPW), c], wsems[b]
            )
        for b in range(_NBUF):
            wd[b].wait()

    return k1


def kernel(x, embedding_table):
    T, J = x.shape
    xp = _build_permute(T, J)(jnp.transpose(x).astype(jnp.int32))
    rr = lax.optimization_barrier(
        jnp.reshape(embedding_table, (_NUM_EMB * _D // 128, 128))
    )
    rt = jnp.reshape(rr, (_NUM_EMB, _D))
    return _build_gather(T, J)(xp, rt)
